# Initial kernel scaffold; baseline (speedup 1.0000x reference)
import jax
import jax.numpy as jnp
from jax.experimental import pallas as pl


def kernel(x, edge_index, edge_feats, Wq1, bq1, Wk1, bk1, Wv1, bv1, We1, Wskip1, bskip1, Wq2, bq2, Wk2, bk2, Wv2, bv2, We2, Wskip2, bskip2):
    raise NotImplementedError("write your pallas kernel here")


# speedup vs baseline: 14.2784x; 14.2784x over previous
import functools
import numpy as np
import jax
import jax.numpy as jnp
from jax import lax
from jax.experimental import pallas as pl
from jax.experimental.pallas import tpu as pltpu
from jax.experimental.pallas import tpu_sc as plsc

_N = 10000
_E = 320000
_D = 128

_NC = 2
_NS = 16
_NW = _NC * _NS
_EPT = _E // _NW
_B = 16
_NCHUNK = _EPT // _B
_NP = 10240
_ND = _NP // 8
_NT = _NP + _ND
_RPT = _NP // _NS
_DPT = _ND // _NS
_TPT = _NT // _NS
_ZC = 16
_F32 = jnp.float32


def _edge_kernel(heads):
  mesh = plsc.VectorSubcoreMesh(core_axis_name="c", subcore_axis_name="s")

  def body(q_hbm, kv_hbm, e_hbm, src_hbm, dst_hbm, acc_out, den_out,
           srcv, dstv, didxv, qb, kvb, eb, resb, exb, zba, accsh,
           sem1, sem2, sem3):
    c = lax.axis_index("c")
    s = lax.axis_index("s")
    w = c * _NS + s
    lane = lax.iota(jnp.int32, 16)
    zero16 = jnp.zeros((16,), _F32)
    bfly_idx = [lane ^ sh for sh in (8, 4, 2, 1)]

    def lsum(v):
      for idx in bfly_idx:
        v = v + jnp.take_along_axis(v, idx, axis=0, mode="promise_in_bounds")
      return v

    def zrow(i, carry):
      for t in range(8):
        zba[i, pl.ds(16 * t, 16)] = zero16
      return carry

    lax.fori_loop(0, _ZC, zrow, 0)

    def zcopy(i, carry):
      r0 = s * _TPT + i * _ZC
      pltpu.sync_copy(zba, accsh.at[pl.ds(r0, _ZC)])
      return carry

    lax.fori_loop(0, _TPT // _ZC, zcopy, 0)
    plsc.subcore_barrier()

    def chunk(i, carry):
      base = w * _EPT + i * _B
      pltpu.sync_copy(src_hbm.at[pl.ds(base, _B)], srcv)
      pltpu.sync_copy(dst_hbm.at[pl.ds(base, _B)], dstv)
      cp1 = pltpu.async_copy(q_hbm.at[dstv], qb, sem1)
      cp2 = pltpu.async_copy(kv_hbm.at[srcv], kvb, sem2)
      cp3 = pltpu.async_copy(e_hbm.at[pl.ds(base, _B)], eb, sem3)
      cp1.wait()
      cp2.wait()
      cp3.wait()

      dv = dstv[...]
      didxv[...] = _NP + lax.shift_right_logical(dv, 3)
      slotv = (dv & 7) * 16

      for j in range(_B):
        if heads == 8:
          den = zero16
          for h in range(8):
            evh = eb[j, pl.ds(16 * h, 16)]
            qh = qb[j, pl.ds(16 * h, 16)]
            kh = kvb[j, pl.ds(16 * h, 16)]
            vh = kvb[j, pl.ds(128 + 16 * h, 16)]
            exh = jnp.exp(lsum(qh * (kh + evh)))
            den = jnp.where(lane == h, exh, den)
            resb[j, pl.ds(16 * h, 16)] = (vh + evh) * exh
        else:
          acc = zero16
          for t in range(8):
            qt = qb[j, pl.ds(16 * t, 16)]
            kt = kvb[j, pl.ds(16 * t, 16)]
            acc = acc + qt * (kt + eb[j, pl.ds(16 * t, 16)])
          ex = jnp.exp(lsum(acc))
          for t in range(8):
            vt = kvb[j, pl.ds(128 + 16 * t, 16)]
            resb[j, pl.ds(16 * t, 16)] = (vt + eb[j, pl.ds(16 * t, 16)]) * ex
          den = jnp.where(lane < 1, ex, 0.0)
        for t in range(8):
          exb[j, pl.ds(16 * t, 16)] = zero16
        exb[j, pl.ds(slotv[j], 16)] = den

      pltpu.sync_copy(resb, accsh.at[dstv], add=True)
      pltpu.sync_copy(exb, accsh.at[didxv], add=True)
      return carry

    lax.fori_loop(0, _NCHUNK, chunk, 0)
    plsc.subcore_barrier()

    pltpu.sync_copy(accsh.at[pl.ds(s * _RPT, _RPT)],
                    acc_out.at[c, pl.ds(s * _RPT, _RPT)])
    pltpu.sync_copy(accsh.at[pl.ds(_NP + s * _DPT, _DPT)],
                    den_out.at[c, pl.ds(s * _DPT, _DPT)])

  return pl.kernel(
      body,
      out_type=[jax.ShapeDtypeStruct((_NC, _NP, 128), _F32),
                jax.ShapeDtypeStruct((_NC, _ND, 128), _F32)],
      mesh=mesh,
      scratch_types=[
          pltpu.VMEM((_B,), jnp.int32),
          pltpu.VMEM((_B,), jnp.int32),
          pltpu.VMEM((_B,), jnp.int32),
          pltpu.VMEM((_B, 128), _F32),
          pltpu.VMEM((_B, 256), _F32),
          pltpu.VMEM((_B, 128), _F32),
          pltpu.VMEM((_B, 128), _F32),
          pltpu.VMEM((_B, 128), _F32),
          pltpu.VMEM((_ZC, 128), _F32),
          pltpu.VMEM_SHARED((_NT, 128), _F32),
          pltpu.SemaphoreType.DMA,
          pltpu.SemaphoreType.DMA,
          pltpu.SemaphoreType.DMA,
      ],
  )


_EDGE8 = _edge_kernel(8)
_EDGE1 = _edge_kernel(1)


def _proj(x, wcat, bcat):
  bn = 2048

  def body(x_ref, w_ref, b_ref, q_ref, kv_ref, s_ref):
    y = jnp.dot(x_ref[...], w_ref[...], preferred_element_type=_F32)
    y = y + b_ref[...]
    q_ref[...] = y[:, :128]
    kv_ref[...] = y[:, 128:384]
    s_ref[...] = y[:, 384:]

  return pl.pallas_call(
      body,
      grid=(_NP // bn,),
      in_specs=[pl.BlockSpec((bn, _D), lambda i: (i, 0)),
                pl.BlockSpec((_D, 512), lambda i: (0, 0)),
                pl.BlockSpec((1, 512), lambda i: (0, 0))],
      out_specs=[pl.BlockSpec((bn, 128), lambda i: (i, 0)),
                 pl.BlockSpec((bn, 256), lambda i: (i, 0)),
                 pl.BlockSpec((bn, 128), lambda i: (i, 0))],
      out_shape=[jax.ShapeDtypeStruct((_NP, 128), _F32),
                 jax.ShapeDtypeStruct((_NP, 256), _F32),
                 jax.ShapeDtypeStruct((_NP, 128), _F32)],
  )(x, wcat, bcat)


def _eproj(ea, wecat):
  be = 4000

  def body(a_ref, w_ref, e1_ref, e2_ref):
    y = jnp.dot(a_ref[...], w_ref[...], preferred_element_type=_F32)
    e1_ref[...] = y[:, :128]
    e2_ref[...] = y[:, 128:]

  return pl.pallas_call(
      body,
      grid=(_E // be,),
      in_specs=[pl.BlockSpec((be, 16), lambda i: (i, 0)),
                pl.BlockSpec((16, 256), lambda i: (0, 0))],
      out_specs=[pl.BlockSpec((be, 128), lambda i: (i, 0)),
                 pl.BlockSpec((be, 128), lambda i: (i, 0))],
      out_shape=[jax.ShapeDtypeStruct((_E, 128), _F32),
                 jax.ShapeDtypeStruct((_E, 128), _F32)],
  )(ea, wecat)


def _mid(acc, den, skip, wcat, bcat, rmat):
  bn = 2048

  def body(acc_ref, den_ref, skip_ref, w_ref, b_ref, r_ref,
           q_ref, kv_ref, s_ref):
    a = acc_ref[0] + acc_ref[1]
    d = den_ref[0] + den_ref[1]
    dexp = jnp.dot(d, r_ref[...], preferred_element_type=_F32)
    h = jnp.maximum(a / (dexp + 1e-16) + skip_ref[...], 0.0)
    y = jnp.dot(h, w_ref[...], preferred_element_type=_F32) + b_ref[...]
    q_ref[...] = y[:, :128]
    kv_ref[...] = y[:, 128:384]
    s_ref[...] = y[:, 384:]

  return pl.pallas_call(
      body,
      grid=(_NP // bn,),
      in_specs=[pl.BlockSpec((_NC, bn, 128), lambda i: (0, i, 0)),
                pl.BlockSpec((_NC, bn, 16), lambda i: (0, i, 0)),
                pl.BlockSpec((bn, 128), lambda i: (i, 0)),
                pl.BlockSpec((_D, 512), lambda i: (0, 0)),
                pl.BlockSpec((1, 512), lambda i: (0, 0)),
                pl.BlockSpec((16, 128), lambda i: (0, 0))],
      out_specs=[pl.BlockSpec((bn, 128), lambda i: (i, 0)),
                 pl.BlockSpec((bn, 256), lambda i: (i, 0)),
                 pl.BlockSpec((bn, 128), lambda i: (i, 0))],
      out_shape=[jax.ShapeDtypeStruct((_NP, 128), _F32),
                 jax.ShapeDtypeStruct((_NP, 256), _F32),
                 jax.ShapeDtypeStruct((_NP, 128), _F32)],
  )(acc, den, skip, wcat, bcat, rmat)


def _final(acc, den, skip, rmat):
  bn = 2048

  def body(acc_ref, den_ref, skip_ref, r_ref, o_ref):
    a = acc_ref[0] + acc_ref[1]
    d = den_ref[0] + den_ref[1]
    dexp = jnp.dot(d, r_ref[...], preferred_element_type=_F32)
    o_ref[...] = jnp.maximum(a / (dexp + 1e-16) + skip_ref[...], 0.0)

  return pl.pallas_call(
      body,
      grid=(_NP // bn,),
      in_specs=[pl.BlockSpec((_NC, bn, 128), lambda i: (0, i, 0)),
                pl.BlockSpec((_NC, bn, 16), lambda i: (0, i, 0)),
                pl.BlockSpec((bn, 128), lambda i: (i, 0)),
                pl.BlockSpec((16, 128), lambda i: (0, 0))],
      out_specs=pl.BlockSpec((bn, 128), lambda i: (i, 0)),
      out_shape=jax.ShapeDtypeStruct((_NP, 128), _F32),
  )(acc, den, skip, rmat)


_R1 = np.zeros((16, 128), np.float32)
for _h in range(8):
  _R1[_h, 16 * _h:16 * _h + 16] = 1.0
_R2 = np.zeros((16, 128), np.float32)
_R2[0, :] = 1.0


def kernel(x, edge_index, edge_feats,
           Wq1, bq1, Wk1, bk1, Wv1, bv1, We1, Wskip1, bskip1,
           Wq2, bq2, Wk2, bk2, Wv2, bv2, We2, Wskip2, bskip2):
  src = edge_index[0]
  dst = edge_index[1]
  s1 = 0.25
  s2 = float(1.0 / np.sqrt(128.0))
  w1cat = jnp.concatenate([Wq1 * s1, Wk1, Wv1, Wskip1], axis=1)
  b1cat = jnp.concatenate([bq1 * s1, bk1, bv1, bskip1]).reshape(1, 512)
  w2cat = jnp.concatenate([Wq2 * s2, Wk2, Wv2, Wskip2], axis=1)
  b2cat = jnp.concatenate([bq2 * s2, bk2, bv2, bskip2]).reshape(1, 512)
  wecat = jnp.concatenate([We1, We2], axis=1)
  r1 = jnp.asarray(_R1)
  r2 = jnp.asarray(_R2)

  xp = jnp.concatenate(
      [x, jnp.zeros((_NP - _N, _D), _F32)], axis=0)
  q1, kv1, skip1 = _proj(xp, w1cat, b1cat)
  e1, e2 = _eproj(edge_feats, wecat)
  acc1, den1 = _EDGE8(q1, kv1, e1, src, dst)
  den1 = den1.reshape(_NC, _NP, 16)
  q2, kv2, skip2 = _mid(acc1, den1, skip1, w2cat, b2cat, r1)
  acc2, den2 = _EDGE1(q2, kv2, e2, src, dst)
  den2 = den2.reshape(_NC, _NP, 16)
  return _final(acc2, den2, skip2, r2)[:_N]
